# baseline (device time: 12134 ns/iter reference)
import jax
import jax.numpy as jnp
from jax import lax
from jax.experimental import pallas as pl
from jax.experimental.pallas import tpu as pltpu

N_PLANE = 4


def kernel(x, dy, gamma):
    m, d = x.shape
    m_loc = m // 2

    def body(x_ref, dy_ref, gamma_ref, out_ref,
             x_vmem, dy_vmem, comm_ref, copy_sems, send_sems, recv_sems):
        my_x = lax.axis_index("x")
        my_y = lax.axis_index("y")
        my_z = lax.axis_index("z")
        q = 2 * my_x + my_y

        cp_x = pltpu.make_async_copy(
            x_ref.at[pl.ds(my_y * m_loc, m_loc), :], x_vmem, copy_sems.at[0])
        cp_dy = pltpu.make_async_copy(
            dy_ref.at[pl.ds(my_y * m_loc, m_loc), :], dy_vmem, copy_sems.at[1])
        cp_x.start()
        cp_dy.start()

        barrier_sem = pltpu.get_barrier_semaphore()
        for k in range(1, N_PLANE):
            p = (q + k) % N_PLANE
            pid = (p // 2, p % 2, my_z)
            pl.semaphore_signal(
                barrier_sem, inc=1,
                device_id=pid, device_id_type=pl.DeviceIdType.MESH,
            )
        pl.semaphore_wait(barrier_sem, N_PLANE - 1)

        cp_x.wait()
        cp_dy.wait()

        xv = x_vmem[:, :].astype(jnp.float32)
        dyv = dy_vmem[:, :].astype(jnp.float32)
        mu = jnp.mean(xv, axis=1, keepdims=True)
        var = jnp.mean((xv - mu) * (xv - mu), axis=1, keepdims=True)
        rstd = lax.rsqrt(var + 1e-5)
        xhat = (xv - mu) * rstd
        dgamma = jnp.sum(dyv * xhat, axis=0, keepdims=True)
        dbeta = jnp.sum(dyv, axis=0, keepdims=True)
        comm_ref[q, :, :] = jnp.concatenate([dgamma, dbeta], axis=0)

        rdmas = []
        for k in range(1, N_PLANE):
            p = (q + k) % N_PLANE
            pid = (p // 2, p % 2, my_z)
            rdma = pltpu.make_async_remote_copy(
                src_ref=comm_ref.at[q],
                dst_ref=comm_ref.at[q],
                send_sem=send_sems.at[k],
                recv_sem=recv_sems.at[q],
                device_id=pid,
                device_id_type=pl.DeviceIdType.MESH,
            )
            rdma.start()
            rdmas.append(rdma)

        for k in range(1, N_PLANE):
            p = (q + k) % N_PLANE
            recv = pltpu.make_async_remote_copy(
                src_ref=comm_ref.at[p],
                dst_ref=comm_ref.at[p],
                send_sem=send_sems.at[k],
                recv_sem=recv_sems.at[p],
                device_id=(my_x, my_y, my_z),
                device_id_type=pl.DeviceIdType.MESH,
            )
            recv.wait_recv()

        out_ref[:, :] = (
            (comm_ref[0, :, :] + comm_ref[1, :, :])
            + (comm_ref[2, :, :] + comm_ref[3, :, :])
        )

        for rdma in rdmas:
            rdma.wait_send()

    return pl.pallas_call(
        body,
        out_shape=jax.ShapeDtypeStruct((2, d), jnp.float32),
        in_specs=[
            pl.BlockSpec(memory_space=pl.ANY),
            pl.BlockSpec(memory_space=pl.ANY),
            pl.BlockSpec(memory_space=pl.ANY),
        ],
        out_specs=pl.BlockSpec(memory_space=pltpu.VMEM),
        scratch_shapes=[
            pltpu.VMEM((m // 2, d), jnp.float32),
            pltpu.VMEM((m // 2, d), jnp.float32),
            pltpu.VMEM((N_PLANE, 2, d), jnp.float32),
            pltpu.SemaphoreType.DMA((2,)),
            pltpu.SemaphoreType.DMA((N_PLANE,)),
            pltpu.SemaphoreType.DMA((N_PLANE,)),
        ],
        compiler_params=pltpu.CompilerParams(collective_id=0),
    )(x, dy, gamma)


# device time: 11517 ns/iter; 1.0536x vs baseline; 1.0536x over previous
import jax
import jax.numpy as jnp
from jax import lax
from jax.experimental import pallas as pl
from jax.experimental.pallas import tpu as pltpu


def kernel(x, dy, gamma):
    m, d = x.shape

    def body(x_ref, dy_ref, gamma_ref, out_ref,
             x_vmem, dy_vmem, comm_ref, copy_sems, send_sem, recv_sem):
        my_x = lax.axis_index("x")
        my_y = lax.axis_index("y")
        my_z = lax.axis_index("z")
        partner = (1 - my_x, my_y, my_z)

        cp_x = pltpu.make_async_copy(x_ref, x_vmem, copy_sems.at[0])
        cp_dy = pltpu.make_async_copy(dy_ref, dy_vmem, copy_sems.at[1])
        cp_x.start()
        cp_dy.start()

        barrier_sem = pltpu.get_barrier_semaphore()
        pl.semaphore_signal(
            barrier_sem, inc=1,
            device_id=partner, device_id_type=pl.DeviceIdType.MESH,
        )
        pl.semaphore_wait(barrier_sem, 1)

        cp_x.wait()
        cp_dy.wait()

        xv = x_vmem[:, :].astype(jnp.float32)
        dyv = dy_vmem[:, :].astype(jnp.float32)
        mu = jnp.mean(xv, axis=1, keepdims=True)
        var = jnp.mean((xv - mu) * (xv - mu), axis=1, keepdims=True)
        rstd = lax.rsqrt(var + 1e-5)
        xhat = (xv - mu) * rstd
        dgamma = jnp.sum(dyv * xhat, axis=0, keepdims=True)
        dbeta = jnp.sum(dyv, axis=0, keepdims=True)
        comm_ref[0, :, :] = jnp.concatenate([dgamma, dbeta], axis=0)

        rdma = pltpu.make_async_remote_copy(
            src_ref=comm_ref.at[0],
            dst_ref=comm_ref.at[1],
            send_sem=send_sem,
            recv_sem=recv_sem,
            device_id=partner,
            device_id_type=pl.DeviceIdType.MESH,
        )
        rdma.start()
        rdma.wait_recv()

        out_ref[:, :] = comm_ref[0, :, :] + comm_ref[1, :, :]

        rdma.wait_send()

    return pl.pallas_call(
        body,
        out_shape=jax.ShapeDtypeStruct((2, d), jnp.float32),
        in_specs=[
            pl.BlockSpec(memory_space=pl.ANY),
            pl.BlockSpec(memory_space=pl.ANY),
            pl.BlockSpec(memory_space=pl.ANY),
        ],
        out_specs=pl.BlockSpec(memory_space=pltpu.VMEM),
        scratch_shapes=[
            pltpu.VMEM((m, d), jnp.float32),
            pltpu.VMEM((m, d), jnp.float32),
            pltpu.VMEM((2, 2, d), jnp.float32),
            pltpu.SemaphoreType.DMA((2,)),
            pltpu.SemaphoreType.DMA,
            pltpu.SemaphoreType.DMA,
        ],
        compiler_params=pltpu.CompilerParams(collective_id=0),
    )(x, dy, gamma)
